# baseline (speedup 1.0000x reference)
import functools

import jax
import jax.numpy as jnp
from jax import lax
from jax.experimental import pallas as pl
from jax.experimental.pallas import tpu as pltpu
from jax.experimental.pallas import tpu_sc as plsc

DD = 768
EE = 8
HH = 3072
TOKS = 2048

TT = 768
NT_MAX = TOKS * 2 // TT + EE
NPAD = NT_MAX * TT
HT_SZ = 1024
HTC = HH // HT_SZ


def _router_kernel(x_ref, gw_ref, d0_ref, d1_ref, w0_ref, w1_ref,
                   info_ref, aux_ref, x16_ref):
    x = x_ref[...]
    x16_ref[...] = x.astype(jnp.bfloat16)
    gw = gw_ref[...]
    logits = jax.lax.dot_general(
        x, gw, (((1,), (1,)), ((), ())), preferred_element_type=jnp.float32)
    col = jax.lax.broadcasted_iota(jnp.int32, (TOKS, EE), 1)
    m1 = jnp.max(logits, axis=1, keepdims=True)
    am1 = jnp.min(jnp.where(logits == m1, col, EE), axis=1, keepdims=True)
    sel1 = col == am1
    l2 = jnp.where(sel1, -jnp.inf, logits)
    m2 = jnp.max(l2, axis=1, keepdims=True)
    am2 = jnp.min(jnp.where(l2 == m2, col, EE), axis=1, keepdims=True)
    sel2 = col == am2
    e21 = jnp.exp(m2 - m1)
    wa = 1.0 / (1.0 + e21)
    w0_ref[...] = wa
    w1_ref[...] = e21 * wa

    one0 = jnp.where(sel1, 1.0, 0.0)
    one1 = jnp.where(sel2, 1.0, 0.0)
    r = jax.lax.broadcasted_iota(jnp.int32, (TOKS, TOKS), 0)
    c = jax.lax.broadcasted_iota(jnp.int32, (TOKS, TOKS), 1)
    tril = jnp.where(r > c, 1.0, 0.0)
    cum0 = jax.lax.dot_general(
        tril, one0, (((1,), (0,)), ((), ())), preferred_element_type=jnp.float32)
    cum1 = jax.lax.dot_general(
        tril, one1, (((1,), (0,)), ((), ())), preferred_element_type=jnp.float32)
    cnt0 = jnp.sum(one0, axis=0, keepdims=True)
    cnt = cnt0 + jnp.sum(one1, axis=0, keepdims=True)
    jlev = jax.lax.broadcasted_iota(jnp.int32, (NT_MAX, EE), 0)
    tiles = jnp.sum(jnp.where(cnt > (jlev * TT).astype(jnp.float32), 1.0, 0.0),
                    axis=0, keepdims=True)
    pc = TT * tiles
    i8 = jax.lax.broadcasted_iota(jnp.int32, (EE, EE), 0)
    j8 = jax.lax.broadcasted_iota(jnp.int32, (EE, EE), 1)
    excl = jnp.where(i8 < j8, 1.0, 0.0)
    off = jax.lax.dot_general(
        pc, excl, (((1,), (0,)), ((), ())), preferred_element_type=jnp.float32)
    d0 = jnp.sum(jnp.where(sel1, cum0 + off, 0.0), axis=1, keepdims=True)
    d1 = jnp.sum(jnp.where(sel2, cum1 + off + cnt0, 0.0), axis=1, keepdims=True)
    d0_ref[...] = d0.astype(jnp.int32)
    d1_ref[...] = d1.astype(jnp.int32)
    ti = jax.lax.broadcasted_iota(jnp.int32, (NT_MAX + 1, EE), 0)
    cover = jnp.where(off <= (ti * TT).astype(jnp.float32), 1.0, 0.0)
    eid = jnp.sum(cover, axis=1, keepdims=True) - 1.0
    ntiles = jnp.sum(tiles, axis=1, keepdims=True)
    ri = jax.lax.broadcasted_iota(jnp.int32, (NT_MAX + 1, 1), 0)
    info_ref[...] = jnp.where(ri == NT_MAX, ntiles, eid).astype(jnp.int32)
    z = jnp.exp(logits - m1)
    p = z / jnp.sum(z, axis=1, keepdims=True)
    aux_ref[0, 0] = (float(EE) / (TOKS * TOKS)) * jnp.sum(
        jnp.sum(one0 + one1, axis=0, keepdims=True)
        * jnp.sum(p, axis=0, keepdims=True))


def _run_router(x_flat, gate_w):
    return pl.pallas_call(
        _router_kernel,
        out_shape=(
            jax.ShapeDtypeStruct((TOKS, 1), jnp.int32),
            jax.ShapeDtypeStruct((TOKS, 1), jnp.int32),
            jax.ShapeDtypeStruct((TOKS, 1), jnp.float32),
            jax.ShapeDtypeStruct((TOKS, 1), jnp.float32),
            jax.ShapeDtypeStruct((NT_MAX + 1, 1), jnp.int32),
            jax.ShapeDtypeStruct((1, 1), jnp.float32),
            jax.ShapeDtypeStruct((TOKS, DD), jnp.bfloat16),
        ),
        in_specs=[
            pl.BlockSpec(memory_space=pltpu.VMEM),
            pl.BlockSpec(memory_space=pltpu.VMEM),
        ],
        out_specs=(
            pl.BlockSpec(memory_space=pltpu.VMEM),
            pl.BlockSpec(memory_space=pltpu.VMEM),
            pl.BlockSpec(memory_space=pltpu.VMEM),
            pl.BlockSpec(memory_space=pltpu.VMEM),
            pl.BlockSpec(memory_space=pltpu.VMEM),
            pl.BlockSpec(memory_space=pltpu.SMEM),
            pl.BlockSpec(memory_space=pltpu.VMEM),
        ),
    )(x_flat, gate_w)


def _run_dispatch(x_flat, d0, d1):
    info = plsc.get_sparse_core_info()
    nc, ns = info.num_cores, info.num_subcores
    chunk = TOKS // (nc * ns)

    @functools.partial(
        pl.kernel,
        mesh=plsc.VectorSubcoreMesh(core_axis_name="c", subcore_axis_name="s"),
        out_type=jax.ShapeDtypeStruct((NPAD, DD // 2), jnp.float32),
        scratch_types=[
            pltpu.VMEM((chunk,), jnp.int32),
            pltpu.VMEM((chunk,), jnp.int32),
            pltpu.VMEM((chunk, DD // 2), jnp.float32),
            pltpu.SemaphoreType.DMA,
            pltpu.SemaphoreType.DMA,
        ],
    )
    def k(x_hbm, d0_hbm, d1_hbm, xs_hbm, i0_v, i1_v, rows_v, s0, s1):
        wid = lax.axis_index("s") * nc + lax.axis_index("c")
        base = wid * chunk
        pltpu.sync_copy(d0_hbm.at[pl.ds(base, chunk)], i0_v)
        pltpu.sync_copy(d1_hbm.at[pl.ds(base, chunk)], i1_v)
        pltpu.sync_copy(x_hbm.at[pl.ds(base, chunk)], rows_v)
        c0 = pltpu.async_copy(rows_v, xs_hbm.at[i0_v], s0)
        c1 = pltpu.async_copy(rows_v, xs_hbm.at[i1_v], s1)
        c0.wait()
        c1.wait()

    return k(x_flat, d0, d1)


def _ffn_kernel(info_ref, xs_ref, f1g_ref, f1v_ref, b1g_ref, b1v_ref,
                f2_ref, b2_ref, o_ref, acc_ref):
    ht = pl.program_id(0)
    i = pl.program_id(1)

    @pl.when(i < info_ref[NT_MAX])
    def _():
        x = xs_ref[...]
        g = jax.lax.dot_general(x, f1g_ref[0].astype(jnp.bfloat16),
                                (((1,), (1,)), ((), ())),
                                preferred_element_type=jnp.float32) + b1g_ref[0]
        v = jax.lax.dot_general(x, f1v_ref[0].astype(jnp.bfloat16),
                                (((1,), (1,)), ((), ())),
                                preferred_element_type=jnp.float32) + b1v_ref[0]
        gated = ((g / (1.0 + jnp.exp(-g))) * v).astype(jnp.bfloat16)
        part = jax.lax.dot_general(gated, f2_ref[0].astype(jnp.bfloat16),
                                   (((1,), (1,)), ((), ())),
                                   preferred_element_type=jnp.float32)

        @pl.when(ht == 0)
        def _():
            acc_ref[pl.ds(i * TT, TT), :] = (part + b2_ref[0]).astype(jnp.bfloat16)

        @pl.when((ht != 0) & (ht != HTC - 1))
        def _():
            acc_ref[pl.ds(i * TT, TT), :] = (
                acc_ref[pl.ds(i * TT, TT), :].astype(jnp.float32) + part
            ).astype(jnp.bfloat16)

        @pl.when(ht == HTC - 1)
        def _():
            o_ref[...] = acc_ref[pl.ds(i * TT, TT), :].astype(jnp.float32) + part


def _run_ffn(info, xs, fc1_w, fc1_b, fc2_w, fc2_b):
    grid_spec = pltpu.PrefetchScalarGridSpec(
        num_scalar_prefetch=1,
        grid=(HTC, NT_MAX),
        in_specs=[
            pl.BlockSpec((TT, DD),
                         lambda ht, i, info: (jnp.where(i < info[NT_MAX], i, 0), 0)),
            pl.BlockSpec((1, HT_SZ, DD), lambda ht, i, info: (info[i], ht, 0)),
            pl.BlockSpec((1, HT_SZ, DD),
                         lambda ht, i, info: (info[i], HTC + ht, 0)),
            pl.BlockSpec((1, 1, HT_SZ),
                         lambda ht, i, info: (info[i] * 2 * HTC + ht, 0, 0)),
            pl.BlockSpec((1, 1, HT_SZ),
                         lambda ht, i, info: (info[i] * 2 * HTC + HTC + ht, 0, 0)),
            pl.BlockSpec((1, DD, HT_SZ), lambda ht, i, info: (info[i], 0, ht)),
            pl.BlockSpec((1, 1, DD), lambda ht, i, info: (info[i], 0, 0)),
        ],
        out_specs=pl.BlockSpec(
            (TT, DD), lambda ht, i, info: (jnp.where(ht == HTC - 1, i, 0), 0)),
        scratch_shapes=[pltpu.VMEM((NPAD, DD), jnp.bfloat16)],
    )
    return pl.pallas_call(
        _ffn_kernel,
        grid_spec=grid_spec,
        out_shape=jax.ShapeDtypeStruct((NPAD, DD), jnp.float32),
    )(info, xs, fc1_w, fc1_w,
      fc1_b.reshape(EE * 2 * HTC, 1, HT_SZ), fc1_b.reshape(EE * 2 * HTC, 1, HT_SZ),
      fc2_w, fc2_b.reshape(EE, 1, DD))


def _run_combine(os, d0, d1, w0, w1):
    info = plsc.get_sparse_core_info()
    nc, ns = info.num_cores, info.num_subcores
    chunk = TOKS // (nc * ns)

    @functools.partial(
        pl.kernel,
        mesh=plsc.VectorSubcoreMesh(core_axis_name="c", subcore_axis_name="s"),
        out_type=jax.ShapeDtypeStruct((TOKS, DD), jnp.float32),
        scratch_types=[
            pltpu.VMEM((chunk,), jnp.int32),
            pltpu.VMEM((chunk,), jnp.int32),
            pltpu.VMEM((chunk,), jnp.float32),
            pltpu.VMEM((chunk,), jnp.float32),
            pltpu.VMEM((chunk, DD), jnp.float32),
            pltpu.VMEM((chunk, DD), jnp.float32),
            pltpu.SemaphoreType.DMA,
            pltpu.SemaphoreType.DMA,
        ],
    )
    def k(os_hbm, d0_hbm, d1_hbm, w0_hbm, w1_hbm, y_hbm,
          i0_v, i1_v, w0_v, w1_v, b0_v, b1_v, s0, s1):
        wid = lax.axis_index("s") * nc + lax.axis_index("c")
        base = wid * chunk
        pltpu.sync_copy(d0_hbm.at[pl.ds(base, chunk)], i0_v)
        pltpu.sync_copy(d1_hbm.at[pl.ds(base, chunk)], i1_v)
        pltpu.sync_copy(w0_hbm.at[pl.ds(base, chunk)], w0_v)
        pltpu.sync_copy(w1_hbm.at[pl.ds(base, chunk)], w1_v)
        c0 = pltpu.async_copy(os_hbm.at[i0_v], b0_v, s0)
        c1 = pltpu.async_copy(os_hbm.at[i1_v], b1_v, s1)
        c0.wait()
        c1.wait()

        def group(gg, carry):
            wv0 = w0_v[pl.ds(gg * 16, 16)]
            wv1 = w1_v[pl.ds(gg * 16, 16)]
            for t16 in range(16):
                t = gg * 16 + t16
                wa = jax.lax.squeeze(
                    jax.lax.slice_in_dim(wv0, t16, t16 + 1), (0,))
                wb = jax.lax.squeeze(
                    jax.lax.slice_in_dim(wv1, t16, t16 + 1), (0,))

                def col(cix, c2):
                    sl = pl.ds(cix * 16, 16)
                    b0_v[t, sl] = wa * b0_v[t, sl] + wb * b1_v[t, sl]
                    return c2

                lax.fori_loop(0, DD // 16, col, 0)
            return carry

        lax.fori_loop(0, chunk // 16, group, 0)
        pltpu.sync_copy(b0_v, y_hbm.at[pl.ds(base, chunk)])

    return k(os, d0, d1, w0, w1)


@jax.jit
def kernel(x, gate_w, fc1_w, fc1_b, fc2_w, fc2_b):
    x_flat = x.reshape(TOKS, DD)
    d0, d1, w0, w1, info, aux, x16 = _run_router(x_flat, gate_w)
    d0 = d0.reshape(TOKS)
    d1 = d1.reshape(TOKS)
    xpk = jax.lax.bitcast_convert_type(
        x16.reshape(TOKS, DD // 2, 2), jnp.float32)
    xs_pk = _run_dispatch(xpk, d0, d1)
    xs16 = jax.lax.bitcast_convert_type(xs_pk, jnp.bfloat16).reshape(NPAD, DD)
    os = _run_ffn(info.reshape(NT_MAX + 1), xs16, fc1_w, fc1_b, fc2_w, fc2_b)
    y = _run_combine(os, d0, d1, w0.reshape(TOKS), w1.reshape(TOKS))
    return y.reshape(x.shape), aux[0, 0]


# speedup vs baseline: 1.9479x; 1.9479x over previous
import functools

import jax
import jax.numpy as jnp
from jax import lax
from jax.experimental import pallas as pl
from jax.experimental.pallas import tpu as pltpu
from jax.experimental.pallas import tpu_sc as plsc

DD = 768
EE = 8
HH = 3072
TOKS = 2048

TT = 768
NT_MAX = TOKS * 2 // TT + EE
NPAD = NT_MAX * TT
HT_SZ = 1024
HTC = HH // HT_SZ


def _router_kernel(x_ref, gw_ref, d0_ref, d1_ref, w0_ref, w1_ref,
                   info_ref, aux_ref):
    x = x_ref[...]
    gw = gw_ref[...]
    logits = jax.lax.dot_general(
        x, gw, (((1,), (1,)), ((), ())), preferred_element_type=jnp.float32)
    col = jax.lax.broadcasted_iota(jnp.int32, (TOKS, EE), 1)
    m1 = jnp.max(logits, axis=1, keepdims=True)
    am1 = jnp.min(jnp.where(logits == m1, col, EE), axis=1, keepdims=True)
    sel1 = col == am1
    l2 = jnp.where(sel1, -jnp.inf, logits)
    m2 = jnp.max(l2, axis=1, keepdims=True)
    am2 = jnp.min(jnp.where(l2 == m2, col, EE), axis=1, keepdims=True)
    sel2 = col == am2
    e21 = jnp.exp(m2 - m1)
    wa = 1.0 / (1.0 + e21)
    w0_ref[...] = wa
    w1_ref[...] = e21 * wa

    one0 = jnp.where(sel1, 1.0, 0.0)
    one1 = jnp.where(sel2, 1.0, 0.0)
    r = jax.lax.broadcasted_iota(jnp.int32, (TOKS, TOKS), 0)
    c = jax.lax.broadcasted_iota(jnp.int32, (TOKS, TOKS), 1)
    tril = jnp.where(r > c, 1.0, 0.0)
    cum0 = jax.lax.dot_general(
        tril, one0, (((1,), (0,)), ((), ())), preferred_element_type=jnp.float32)
    cum1 = jax.lax.dot_general(
        tril, one1, (((1,), (0,)), ((), ())), preferred_element_type=jnp.float32)
    cnt0 = jnp.sum(one0, axis=0, keepdims=True)
    cnt = cnt0 + jnp.sum(one1, axis=0, keepdims=True)
    jlev = jax.lax.broadcasted_iota(jnp.int32, (NT_MAX, EE), 0)
    tiles = jnp.sum(jnp.where(cnt > (jlev * TT).astype(jnp.float32), 1.0, 0.0),
                    axis=0, keepdims=True)
    pc = TT * tiles
    i8 = jax.lax.broadcasted_iota(jnp.int32, (EE, EE), 0)
    j8 = jax.lax.broadcasted_iota(jnp.int32, (EE, EE), 1)
    excl = jnp.where(i8 < j8, 1.0, 0.0)
    off = jax.lax.dot_general(
        pc, excl, (((1,), (0,)), ((), ())), preferred_element_type=jnp.float32)
    d0 = jnp.sum(jnp.where(sel1, cum0 + off, 0.0), axis=1, keepdims=True)
    d1 = jnp.sum(jnp.where(sel2, cum1 + off + cnt0, 0.0), axis=1, keepdims=True)
    d0_ref[...] = d0.astype(jnp.int32)
    d1_ref[...] = d1.astype(jnp.int32)
    ti = jax.lax.broadcasted_iota(jnp.int32, (NT_MAX + 1, EE), 0)
    cover = jnp.where(off <= (ti * TT).astype(jnp.float32), 1.0, 0.0)
    eid = jnp.sum(cover, axis=1, keepdims=True) - 1.0
    ntiles = jnp.sum(tiles, axis=1, keepdims=True)
    ri = jax.lax.broadcasted_iota(jnp.int32, (NT_MAX + 1, 1), 0)
    info_ref[...] = jnp.where(ri == NT_MAX, ntiles, eid).astype(jnp.int32)
    z = jnp.exp(logits - m1)
    p = z / jnp.sum(z, axis=1, keepdims=True)
    aux_ref[0, 0] = (float(EE) / (TOKS * TOKS)) * jnp.sum(
        jnp.sum(one0 + one1, axis=0, keepdims=True)
        * jnp.sum(p, axis=0, keepdims=True))


def _run_router(x_flat, gate_w):
    return pl.pallas_call(
        _router_kernel,
        out_shape=(
            jax.ShapeDtypeStruct((TOKS, 1), jnp.int32),
            jax.ShapeDtypeStruct((TOKS, 1), jnp.int32),
            jax.ShapeDtypeStruct((TOKS, 1), jnp.float32),
            jax.ShapeDtypeStruct((TOKS, 1), jnp.float32),
            jax.ShapeDtypeStruct((NT_MAX + 1, 1), jnp.int32),
            jax.ShapeDtypeStruct((1, 1), jnp.float32),
        ),
        in_specs=[
            pl.BlockSpec(memory_space=pltpu.VMEM),
            pl.BlockSpec(memory_space=pltpu.VMEM),
        ],
        out_specs=(
            pl.BlockSpec(memory_space=pltpu.VMEM),
            pl.BlockSpec(memory_space=pltpu.VMEM),
            pl.BlockSpec(memory_space=pltpu.VMEM),
            pl.BlockSpec(memory_space=pltpu.VMEM),
            pl.BlockSpec(memory_space=pltpu.VMEM),
            pl.BlockSpec(memory_space=pltpu.SMEM),
        ),
    )(x_flat, gate_w)


def _run_dispatch(x_flat, d0, d1):
    info = plsc.get_sparse_core_info()
    nc, ns = info.num_cores, info.num_subcores
    chunk = TOKS // (nc * ns)

    @functools.partial(
        pl.kernel,
        mesh=plsc.VectorSubcoreMesh(core_axis_name="c", subcore_axis_name="s"),
        out_type=jax.ShapeDtypeStruct((NPAD, DD), jnp.float32),
        scratch_types=[
            pltpu.VMEM((chunk,), jnp.int32),
            pltpu.VMEM((chunk,), jnp.int32),
            pltpu.VMEM((chunk, DD), jnp.float32),
            pltpu.SemaphoreType.DMA,
            pltpu.SemaphoreType.DMA,
        ],
    )
    def k(x_hbm, d0_hbm, d1_hbm, xs_hbm, i0_v, i1_v, rows_v, s0, s1):
        wid = lax.axis_index("s") * nc + lax.axis_index("c")
        base = wid * chunk
        pltpu.sync_copy(d0_hbm.at[pl.ds(base, chunk)], i0_v)
        pltpu.sync_copy(d1_hbm.at[pl.ds(base, chunk)], i1_v)
        pltpu.sync_copy(x_hbm.at[pl.ds(base, chunk)], rows_v)
        c0 = pltpu.async_copy(rows_v, xs_hbm.at[i0_v], s0)
        c1 = pltpu.async_copy(rows_v, xs_hbm.at[i1_v], s1)
        c0.wait()
        c1.wait()

    return k(x_flat, d0, d1)


def _ffn_kernel(info_ref, xs_ref, f1g_ref, f1v_ref, b1g_ref, b1v_ref,
                f2_ref, b2_ref, o_ref, acc_ref):
    ht = pl.program_id(0)
    i = pl.program_id(1)

    @pl.when(i < info_ref[NT_MAX])
    def _():
        x = xs_ref[...].astype(jnp.bfloat16)
        g = jax.lax.dot_general(x, f1g_ref[0].astype(jnp.bfloat16),
                                (((1,), (1,)), ((), ())),
                                preferred_element_type=jnp.float32) + b1g_ref[0]
        v = jax.lax.dot_general(x, f1v_ref[0].astype(jnp.bfloat16),
                                (((1,), (1,)), ((), ())),
                                preferred_element_type=jnp.float32) + b1v_ref[0]
        gated = ((g / (1.0 + jnp.exp(-g))) * v).astype(jnp.bfloat16)
        part = jax.lax.dot_general(gated, f2_ref[0].astype(jnp.bfloat16),
                                   (((1,), (1,)), ((), ())),
                                   preferred_element_type=jnp.float32)

        @pl.when(ht == 0)
        def _():
            acc_ref[pl.ds(i * TT, TT), :] = (part + b2_ref[0]).astype(jnp.bfloat16)

        @pl.when((ht != 0) & (ht != HTC - 1))
        def _():
            acc_ref[pl.ds(i * TT, TT), :] = (
                acc_ref[pl.ds(i * TT, TT), :].astype(jnp.float32) + part
            ).astype(jnp.bfloat16)

        @pl.when(ht == HTC - 1)
        def _():
            o_ref[...] = acc_ref[pl.ds(i * TT, TT), :].astype(jnp.float32) + part


def _run_ffn(info, xs, fc1_w, fc1_b, fc2_w, fc2_b):
    grid_spec = pltpu.PrefetchScalarGridSpec(
        num_scalar_prefetch=1,
        grid=(HTC, NT_MAX),
        in_specs=[
            pl.BlockSpec((TT, DD),
                         lambda ht, i, info: (jnp.where(i < info[NT_MAX], i, 0), 0)),
            pl.BlockSpec((1, HT_SZ, DD), lambda ht, i, info: (info[i], ht, 0)),
            pl.BlockSpec((1, HT_SZ, DD),
                         lambda ht, i, info: (info[i], HTC + ht, 0)),
            pl.BlockSpec((1, 1, HT_SZ),
                         lambda ht, i, info: (info[i] * 2 * HTC + ht, 0, 0)),
            pl.BlockSpec((1, 1, HT_SZ),
                         lambda ht, i, info: (info[i] * 2 * HTC + HTC + ht, 0, 0)),
            pl.BlockSpec((1, DD, HT_SZ), lambda ht, i, info: (info[i], 0, ht)),
            pl.BlockSpec((1, 1, DD), lambda ht, i, info: (info[i], 0, 0)),
        ],
        out_specs=pl.BlockSpec(
            (TT, DD), lambda ht, i, info: (jnp.where(ht == HTC - 1, i, 0), 0)),
        scratch_shapes=[pltpu.VMEM((NPAD, DD), jnp.bfloat16)],
    )
    return pl.pallas_call(
        _ffn_kernel,
        grid_spec=grid_spec,
        out_shape=jax.ShapeDtypeStruct((NPAD, DD), jnp.float32),
    )(info, xs, fc1_w, fc1_w,
      fc1_b.reshape(EE * 2 * HTC, 1, HT_SZ), fc1_b.reshape(EE * 2 * HTC, 1, HT_SZ),
      fc2_w, fc2_b.reshape(EE, 1, DD))


def _run_combine(os, d0, d1, w0, w1):
    info = plsc.get_sparse_core_info()
    nc, ns = info.num_cores, info.num_subcores
    chunk = TOKS // (nc * ns)

    @functools.partial(
        pl.kernel,
        mesh=plsc.VectorSubcoreMesh(core_axis_name="c", subcore_axis_name="s"),
        out_type=jax.ShapeDtypeStruct((TOKS, DD), jnp.float32),
        scratch_types=[
            pltpu.VMEM((chunk,), jnp.int32),
            pltpu.VMEM((chunk,), jnp.int32),
            pltpu.VMEM((chunk,), jnp.float32),
            pltpu.VMEM((chunk,), jnp.float32),
            pltpu.VMEM((chunk, DD), jnp.float32),
            pltpu.VMEM((chunk, DD), jnp.float32),
            pltpu.SemaphoreType.DMA,
            pltpu.SemaphoreType.DMA,
        ],
    )
    def k(os_hbm, d0_hbm, d1_hbm, w0_hbm, w1_hbm, y_hbm,
          i0_v, i1_v, w0_v, w1_v, b0_v, b1_v, s0, s1):
        wid = lax.axis_index("s") * nc + lax.axis_index("c")
        base = wid * chunk
        pltpu.sync_copy(d0_hbm.at[pl.ds(base, chunk)], i0_v)
        pltpu.sync_copy(d1_hbm.at[pl.ds(base, chunk)], i1_v)
        pltpu.sync_copy(w0_hbm.at[pl.ds(base, chunk)], w0_v)
        pltpu.sync_copy(w1_hbm.at[pl.ds(base, chunk)], w1_v)
        c0 = pltpu.async_copy(os_hbm.at[i0_v], b0_v, s0)
        c1 = pltpu.async_copy(os_hbm.at[i1_v], b1_v, s1)
        c0.wait()
        c1.wait()

        def group(gg, carry):
            wv0 = w0_v[pl.ds(gg * 16, 16)]
            wv1 = w1_v[pl.ds(gg * 16, 16)]
            for t16 in range(16):
                t = gg * 16 + t16
                wa = jax.lax.squeeze(
                    jax.lax.slice_in_dim(wv0, t16, t16 + 1), (0,))
                wb = jax.lax.squeeze(
                    jax.lax.slice_in_dim(wv1, t16, t16 + 1), (0,))

                def col(cix, c2):
                    sl = pl.ds(cix * 16, 16)
                    b0_v[t, sl] = wa * b0_v[t, sl] + wb * b1_v[t, sl]
                    return c2

                lax.fori_loop(0, DD // 16, col, 0)
            return carry

        lax.fori_loop(0, chunk // 16, group, 0)
        pltpu.sync_copy(b0_v, y_hbm.at[pl.ds(base, chunk)])

    return k(os, d0, d1, w0, w1)


@jax.jit
def kernel(x, gate_w, fc1_w, fc1_b, fc2_w, fc2_b):
    x_flat = x.reshape(TOKS, DD)
    d0, d1, w0, w1, info, aux = _run_router(x_flat, gate_w)
    d0 = d0.reshape(TOKS)
    d1 = d1.reshape(TOKS)
    xs = _run_dispatch(x_flat, d0, d1)
    os = _run_ffn(info.reshape(NT_MAX + 1), xs, fc1_w, fc1_b, fc2_w, fc2_b)
    y = _run_combine(os, d0, d1, w0.reshape(TOKS), w1.reshape(TOKS))
    return y.reshape(x.shape), aux[0, 0]


# speedup vs baseline: 1.9923x; 1.0228x over previous
import functools

import jax
import jax.numpy as jnp
from jax import lax
from jax.experimental import pallas as pl
from jax.experimental.pallas import tpu as pltpu
from jax.experimental.pallas import tpu_sc as plsc

DD = 768
EE = 8
HH = 3072
TOKS = 2048

TT = 768
NT_MAX = TOKS * 2 // TT + EE
NPAD = NT_MAX * TT
HT_SZ = 1024
HTC = HH // HT_SZ


def _router_kernel(x_ref, gw_ref, d0_ref, d1_ref, w0_ref, w1_ref,
                   info_ref, aux_ref):
    x = x_ref[...]
    gw = gw_ref[...]
    logits = jax.lax.dot_general(
        x, gw, (((1,), (1,)), ((), ())), preferred_element_type=jnp.float32)
    col = jax.lax.broadcasted_iota(jnp.int32, (TOKS, EE), 1)
    m1 = jnp.max(logits, axis=1, keepdims=True)
    am1 = jnp.min(jnp.where(logits == m1, col, EE), axis=1, keepdims=True)
    sel1 = col == am1
    l2 = jnp.where(sel1, -jnp.inf, logits)
    m2 = jnp.max(l2, axis=1, keepdims=True)
    am2 = jnp.min(jnp.where(l2 == m2, col, EE), axis=1, keepdims=True)
    sel2 = col == am2
    e21 = jnp.exp(m2 - m1)
    wa = 1.0 / (1.0 + e21)
    w0_ref[...] = wa
    w1_ref[...] = e21 * wa

    one0 = jnp.where(sel1, 1.0, 0.0)
    one1 = jnp.where(sel2, 1.0, 0.0)
    r = jax.lax.broadcasted_iota(jnp.int32, (TOKS, TOKS), 0)
    c = jax.lax.broadcasted_iota(jnp.int32, (TOKS, TOKS), 1)
    tril = jnp.where(r > c, 1.0, 0.0)
    cum0 = jax.lax.dot_general(
        tril, one0, (((1,), (0,)), ((), ())), preferred_element_type=jnp.float32)
    cum1 = jax.lax.dot_general(
        tril, one1, (((1,), (0,)), ((), ())), preferred_element_type=jnp.float32)
    cnt0 = jnp.sum(one0, axis=0, keepdims=True)
    cnt = cnt0 + jnp.sum(one1, axis=0, keepdims=True)
    jlev = jax.lax.broadcasted_iota(jnp.int32, (NT_MAX, EE), 0)
    tiles = jnp.sum(jnp.where(cnt > (jlev * TT).astype(jnp.float32), 1.0, 0.0),
                    axis=0, keepdims=True)
    pc = TT * tiles
    i8 = jax.lax.broadcasted_iota(jnp.int32, (EE, EE), 0)
    j8 = jax.lax.broadcasted_iota(jnp.int32, (EE, EE), 1)
    excl = jnp.where(i8 < j8, 1.0, 0.0)
    off = jax.lax.dot_general(
        pc, excl, (((1,), (0,)), ((), ())), preferred_element_type=jnp.float32)
    d0 = jnp.sum(jnp.where(sel1, cum0 + off, 0.0), axis=1, keepdims=True)
    d1 = jnp.sum(jnp.where(sel2, cum1 + off + cnt0, 0.0), axis=1, keepdims=True)
    d0_ref[...] = d0.astype(jnp.int32)
    d1_ref[...] = d1.astype(jnp.int32)
    ti = jax.lax.broadcasted_iota(jnp.int32, (NT_MAX + 1, EE), 0)
    cover = jnp.where(off <= (ti * TT).astype(jnp.float32), 1.0, 0.0)
    eid = jnp.sum(cover, axis=1, keepdims=True) - 1.0
    ntiles = jnp.sum(tiles, axis=1, keepdims=True)
    ri = jax.lax.broadcasted_iota(jnp.int32, (NT_MAX + 1, 1), 0)
    info_ref[...] = jnp.where(ri == NT_MAX, ntiles, eid).astype(jnp.int32)
    z = jnp.exp(logits - m1)
    p = z / jnp.sum(z, axis=1, keepdims=True)
    aux_ref[0, 0] = (float(EE) / (TOKS * TOKS)) * jnp.sum(
        jnp.sum(one0 + one1, axis=0, keepdims=True)
        * jnp.sum(p, axis=0, keepdims=True))


def _run_router(x_flat, gate_w):
    return pl.pallas_call(
        _router_kernel,
        out_shape=(
            jax.ShapeDtypeStruct((TOKS, 1), jnp.int32),
            jax.ShapeDtypeStruct((TOKS, 1), jnp.int32),
            jax.ShapeDtypeStruct((TOKS, 1), jnp.float32),
            jax.ShapeDtypeStruct((TOKS, 1), jnp.float32),
            jax.ShapeDtypeStruct((NT_MAX + 1, 1), jnp.int32),
            jax.ShapeDtypeStruct((1, 1), jnp.float32),
        ),
        in_specs=[
            pl.BlockSpec(memory_space=pltpu.VMEM),
            pl.BlockSpec(memory_space=pltpu.VMEM),
        ],
        out_specs=(
            pl.BlockSpec(memory_space=pltpu.VMEM),
            pl.BlockSpec(memory_space=pltpu.VMEM),
            pl.BlockSpec(memory_space=pltpu.VMEM),
            pl.BlockSpec(memory_space=pltpu.VMEM),
            pl.BlockSpec(memory_space=pltpu.VMEM),
            pl.BlockSpec(memory_space=pltpu.SMEM),
        ),
    )(x_flat, gate_w)


def _run_dispatch(x_flat, d0, d1):
    info = plsc.get_sparse_core_info()
    nc, ns = info.num_cores, info.num_subcores
    chunk = TOKS // (nc * ns)

    @functools.partial(
        pl.kernel,
        mesh=plsc.VectorSubcoreMesh(core_axis_name="c", subcore_axis_name="s"),
        out_type=jax.ShapeDtypeStruct((NPAD, DD), jnp.float32),
        scratch_types=[
            pltpu.VMEM((chunk,), jnp.int32),
            pltpu.VMEM((chunk,), jnp.int32),
            pltpu.VMEM((chunk, DD), jnp.float32),
            pltpu.SemaphoreType.DMA,
            pltpu.SemaphoreType.DMA,
            pltpu.SemaphoreType.DMA,
        ],
    )
    def k(x_hbm, d0_hbm, d1_hbm, xs_hbm, i0_v, i1_v, rows_v, s0, s1, s2):
        wid = lax.axis_index("s") * nc + lax.axis_index("c")
        base = wid * chunk
        a0 = pltpu.async_copy(d0_hbm.at[pl.ds(base, chunk)], i0_v, s0)
        a1 = pltpu.async_copy(d1_hbm.at[pl.ds(base, chunk)], i1_v, s1)
        a2 = pltpu.async_copy(x_hbm.at[pl.ds(base, chunk)], rows_v, s2)
        a0.wait()
        a2.wait()
        c0 = pltpu.async_copy(rows_v, xs_hbm.at[i0_v], s0)
        a1.wait()
        c1 = pltpu.async_copy(rows_v, xs_hbm.at[i1_v], s1)
        c0.wait()
        c1.wait()

    return k(x_flat, d0, d1)


def _ffn_kernel(info_ref, xs_ref, f1g_ref, f1v_ref, b1g_ref, b1v_ref,
                f2_ref, b2_ref, o_ref, acc_ref):
    ht = pl.program_id(0)
    i = pl.program_id(1)

    @pl.when(i < info_ref[NT_MAX])
    def _():
        x = xs_ref[...].astype(jnp.bfloat16)
        g = jax.lax.dot_general(x, f1g_ref[0].astype(jnp.bfloat16),
                                (((1,), (1,)), ((), ())),
                                preferred_element_type=jnp.float32) + b1g_ref[0]
        v = jax.lax.dot_general(x, f1v_ref[0].astype(jnp.bfloat16),
                                (((1,), (1,)), ((), ())),
                                preferred_element_type=jnp.float32) + b1v_ref[0]
        gated = ((g / (1.0 + jnp.exp(-g))) * v).astype(jnp.bfloat16)
        part = jax.lax.dot_general(gated, f2_ref[0].astype(jnp.bfloat16),
                                   (((1,), (1,)), ((), ())),
                                   preferred_element_type=jnp.float32)

        @pl.when(ht == 0)
        def _():
            acc_ref[pl.ds(i * TT, TT), :] = (part + b2_ref[0]).astype(jnp.bfloat16)

        @pl.when((ht != 0) & (ht != HTC - 1))
        def _():
            acc_ref[pl.ds(i * TT, TT), :] = (
                acc_ref[pl.ds(i * TT, TT), :].astype(jnp.float32) + part
            ).astype(jnp.bfloat16)

        @pl.when(ht == HTC - 1)
        def _():
            o_ref[...] = acc_ref[pl.ds(i * TT, TT), :].astype(jnp.float32) + part


def _run_ffn(info, xs, fc1_w, fc1_b, fc2_w, fc2_b):
    grid_spec = pltpu.PrefetchScalarGridSpec(
        num_scalar_prefetch=1,
        grid=(HTC, NT_MAX),
        in_specs=[
            pl.BlockSpec((TT, DD),
                         lambda ht, i, info: (jnp.where(i < info[NT_MAX], i, 0), 0)),
            pl.BlockSpec((1, HT_SZ, DD), lambda ht, i, info: (info[i], ht, 0)),
            pl.BlockSpec((1, HT_SZ, DD),
                         lambda ht, i, info: (info[i], HTC + ht, 0)),
            pl.BlockSpec((1, 1, HT_SZ),
                         lambda ht, i, info: (info[i] * 2 * HTC + ht, 0, 0)),
            pl.BlockSpec((1, 1, HT_SZ),
                         lambda ht, i, info: (info[i] * 2 * HTC + HTC + ht, 0, 0)),
            pl.BlockSpec((1, DD, HT_SZ), lambda ht, i, info: (info[i], 0, ht)),
            pl.BlockSpec((1, 1, DD), lambda ht, i, info: (info[i], 0, 0)),
        ],
        out_specs=pl.BlockSpec(
            (TT, DD),
            lambda ht, i, info: (jnp.where(
                ht == HTC - 1,
                jnp.minimum(i, jnp.maximum(info[NT_MAX] - 1, 0)), 0), 0)),
        scratch_shapes=[pltpu.VMEM((NPAD, DD), jnp.bfloat16)],
    )
    return pl.pallas_call(
        _ffn_kernel,
        grid_spec=grid_spec,
        out_shape=jax.ShapeDtypeStruct((NPAD, DD), jnp.float32),
    )(info, xs, fc1_w, fc1_w,
      fc1_b.reshape(EE * 2 * HTC, 1, HT_SZ), fc1_b.reshape(EE * 2 * HTC, 1, HT_SZ),
      fc2_w, fc2_b.reshape(EE, 1, DD))


def _run_combine(os, d0, d1, w0, w1):
    info = plsc.get_sparse_core_info()
    nc, ns = info.num_cores, info.num_subcores
    chunk = TOKS // (nc * ns)

    @functools.partial(
        pl.kernel,
        mesh=plsc.VectorSubcoreMesh(core_axis_name="c", subcore_axis_name="s"),
        out_type=jax.ShapeDtypeStruct((TOKS, DD), jnp.float32),
        scratch_types=[
            pltpu.VMEM((chunk,), jnp.int32),
            pltpu.VMEM((chunk,), jnp.int32),
            pltpu.VMEM((chunk,), jnp.float32),
            pltpu.VMEM((chunk,), jnp.float32),
            pltpu.VMEM((chunk, DD), jnp.float32),
            pltpu.VMEM((chunk, DD), jnp.float32),
            pltpu.SemaphoreType.DMA,
            pltpu.SemaphoreType.DMA,
        ],
    )
    def k(os_hbm, d0_hbm, d1_hbm, w0_hbm, w1_hbm, y_hbm,
          i0_v, i1_v, w0_v, w1_v, b0_v, b1_v, s0, s1):
        wid = lax.axis_index("s") * nc + lax.axis_index("c")
        base = wid * chunk
        a0 = pltpu.async_copy(d0_hbm.at[pl.ds(base, chunk)], i0_v, s0)
        a1 = pltpu.async_copy(d1_hbm.at[pl.ds(base, chunk)], i1_v, s1)
        pltpu.sync_copy(w0_hbm.at[pl.ds(base, chunk)], w0_v)
        pltpu.sync_copy(w1_hbm.at[pl.ds(base, chunk)], w1_v)
        a0.wait()
        c0 = pltpu.async_copy(os_hbm.at[i0_v], b0_v, s0)
        a1.wait()
        c1 = pltpu.async_copy(os_hbm.at[i1_v], b1_v, s1)
        c0.wait()
        c1.wait()

        def group(gg, carry):
            wv0 = w0_v[pl.ds(gg * 16, 16)]
            wv1 = w1_v[pl.ds(gg * 16, 16)]
            for t16 in range(16):
                t = gg * 16 + t16
                wa = jax.lax.squeeze(
                    jax.lax.slice_in_dim(wv0, t16, t16 + 1), (0,))
                wb = jax.lax.squeeze(
                    jax.lax.slice_in_dim(wv1, t16, t16 + 1), (0,))

                def col(cix, c2):
                    sl = pl.ds(cix * 16, 16)
                    b0_v[t, sl] = wa * b0_v[t, sl] + wb * b1_v[t, sl]
                    return c2

                lax.fori_loop(0, DD // 16, col, 0)
            return carry

        lax.fori_loop(0, chunk // 16, group, 0)
        pltpu.sync_copy(b0_v, y_hbm.at[pl.ds(base, chunk)])

    return k(os, d0, d1, w0, w1)


@jax.jit
def kernel(x, gate_w, fc1_w, fc1_b, fc2_w, fc2_b):
    x_flat = x.reshape(TOKS, DD)
    d0, d1, w0, w1, info, aux = _run_router(x_flat, gate_w)
    d0 = d0.reshape(TOKS)
    d1 = d1.reshape(TOKS)
    xs = _run_dispatch(x_flat, d0, d1)
    os = _run_ffn(info.reshape(NT_MAX + 1), xs, fc1_w, fc1_b, fc2_w, fc2_b)
    y = _run_combine(os, d0, d1, w0.reshape(TOKS), w1.reshape(TOKS))
    return y.reshape(x.shape), aux[0, 0]
